# baseline (speedup 1.0000x reference)
import functools

import jax
import jax.numpy as jnp
from jax import lax
from jax.experimental import pallas as pl
from jax.experimental.pallas import tpu as pltpu
from jax.experimental.pallas import tpu_sc as plsc

_HID = 768
_NEXP = 8
_LANES = 16
_NCORES = 2
_NSUB = 16
_NWORK = _NCORES * _NSUB

_TC_BLK = 2048


def _logits_body(x_ref, w_ref, out_ref):
    out_ref[...] = lax.dot_general(
        x_ref[...], w_ref[...],
        dimension_numbers=(((1,), (0,)), ((), ())),
        preferred_element_type=jnp.float32)


def _compute_logits(flat, w_t):
    n_tok = flat.shape[0]
    return pl.pallas_call(
        _logits_body,
        grid=(n_tok // _TC_BLK,),
        in_specs=[
            pl.BlockSpec((_TC_BLK, _HID), lambda i: (i, 0)),
            pl.BlockSpec((_HID, _NEXP), lambda i: (0, 0)),
        ],
        out_specs=pl.BlockSpec((_TC_BLK, _NEXP), lambda i: (i, 0)),
        out_shape=jax.ShapeDtypeStruct((n_tok, _NEXP), jnp.float32),
    )(flat, w_t)


def _make_router(n_tok):
    tpw = n_tok // _NWORK

    @functools.partial(
        pl.kernel,
        mesh=plsc.VectorSubcoreMesh(core_axis_name="c", subcore_axis_name="s"),
        compiler_params=pltpu.CompilerParams(needs_layout_passes=False),
        out_type=[
            jax.ShapeDtypeStruct((n_tok * 2,), jnp.float32),
            jax.ShapeDtypeStruct((n_tok * 2,), jnp.int32),
        ],
        scratch_types=[
            pltpu.VMEM((tpw * _NEXP,), jnp.float32),
            pltpu.VMEM((tpw * 2,), jnp.float32),
            pltpu.VMEM((tpw * 2,), jnp.int32),
        ],
    )
    def router(logits_hbm, w_hbm, i_hbm, lg_v, w_v, i_v):
        wid = lax.axis_index("s") * _NCORES + lax.axis_index("c")
        base = wid * tpw
        pltpu.sync_copy(logits_hbm.at[pl.ds(base * _NEXP, tpw * _NEXP)], lg_v)
        lane = lax.iota(jnp.int32, _LANES)

        def group(g, carry):
            row = g * _LANES + lane
            row8 = row * _NEXP
            ls = [
                plsc.load_gather(lg_v, [row8 + e])
                for e in range(_NEXP)
            ]
            m1 = ls[0]
            i1 = jnp.zeros((_LANES,), jnp.int32)
            m2 = jnp.full((_LANES,), -jnp.inf, jnp.float32)
            i2 = jnp.zeros((_LANES,), jnp.int32)
            for e in range(1, _NEXP):
                le = ls[e]
                ev = jnp.full((_LANES,), e, jnp.int32)
                gt1 = le > m1
                gt2 = le > m2
                m2n = jnp.where(gt1, m1, jnp.where(gt2, le, m2))
                i2n = jnp.where(gt1, i1, jnp.where(gt2, ev, i2))
                m1 = jnp.where(gt1, le, m1)
                i1 = jnp.where(gt1, ev, i1)
                m2 = m2n
                i2 = i2n
            z = jnp.exp(m2 - m1)
            s = 1.0 + z
            w1 = 1.0 / s
            w2 = z / s
            row2 = row * 2
            plsc.store_scatter(w_v, [row2], w1)
            plsc.store_scatter(w_v, [row2 + 1], w2)
            plsc.store_scatter(i_v, [row2], i1)
            plsc.store_scatter(i_v, [row2 + 1], i2)
            return carry

        lax.fori_loop(0, tpw // _LANES, group, 0)
        pltpu.sync_copy(w_v, w_hbm.at[pl.ds(base * 2, tpw * 2)])
        pltpu.sync_copy(i_v, i_hbm.at[pl.ds(base * 2, tpw * 2)])

    return router


def kernel(hidden_states, gate_w):
    b, s, h = hidden_states.shape
    flat = hidden_states.reshape(-1, h)
    n_tok = flat.shape[0]
    logits = _compute_logits(flat, gate_w.T)
    w_flat, i_flat = _make_router(n_tok)(logits.reshape(-1))
    return w_flat.reshape(n_tok, 2), i_flat.reshape(n_tok, 2), logits


# speedup vs baseline: 1.0433x; 1.0433x over previous
import functools

import jax
import jax.numpy as jnp
from jax import lax
from jax.experimental import pallas as pl
from jax.experimental.pallas import tpu as pltpu
from jax.experimental.pallas import tpu_sc as plsc

_HID = 768
_NEXP = 8
_LANES = 16
_NCORES = 2
_NSUB = 16
_NWORK = _NCORES * _NSUB
_CHUNK = 512

_TC_BLK = 2048


def _logits_body(x_ref, w_ref, out_ref):
    out_ref[...] = lax.dot_general(
        x_ref[...], w_ref[...],
        dimension_numbers=(((1,), (0,)), ((), ())),
        preferred_element_type=jnp.float32)


def _compute_logits(flat, w_t):
    n_tok = flat.shape[0]
    return pl.pallas_call(
        _logits_body,
        grid=(n_tok // _TC_BLK,),
        in_specs=[
            pl.BlockSpec((_TC_BLK, _HID), lambda i: (i, 0)),
            pl.BlockSpec((_HID, _NEXP), lambda i: (0, 0)),
        ],
        out_specs=pl.BlockSpec((_TC_BLK, _NEXP), lambda i: (i, 0)),
        out_shape=jax.ShapeDtypeStruct((n_tok, _NEXP), jnp.float32),
    )(flat, w_t)


def _make_router(n_tok):
    tpw = n_tok // _NWORK

    @functools.partial(
        pl.kernel,
        mesh=plsc.VectorSubcoreMesh(core_axis_name="c", subcore_axis_name="s"),
        compiler_params=pltpu.CompilerParams(needs_layout_passes=False),
        out_type=[
            jax.ShapeDtypeStruct((n_tok * 2,), jnp.float32),
            jax.ShapeDtypeStruct((n_tok * 2,), jnp.int32),
        ],
        scratch_types=[
            pltpu.VMEM((_CHUNK, _NEXP), jnp.float32),
            pltpu.VMEM((tpw * 2,), jnp.float32),
            pltpu.VMEM((tpw * 2,), jnp.int32),
        ],
    )
    def router(logits_hbm, w_hbm, i_hbm, lg_v, w_v, i_v):
        wid = lax.axis_index("s") * _NCORES + lax.axis_index("c")
        base = wid * tpw
        lane = lax.iota(jnp.int32, _LANES)

        def chunk(c, carry):
            pltpu.sync_copy(logits_hbm.at[pl.ds(base + c * _CHUNK, _CHUNK)],
                            lg_v)

            def group(g, _):
                row = g * _LANES + lane
                ls = [
                    plsc.load_gather(
                        lg_v, [row, jnp.full((_LANES,), e, jnp.int32)])
                    for e in range(_NEXP)
                ]
                m1 = ls[0]
                i1 = jnp.zeros((_LANES,), jnp.int32)
                m2 = jnp.full((_LANES,), -jnp.inf, jnp.float32)
                i2 = jnp.zeros((_LANES,), jnp.int32)
                for e in range(1, _NEXP):
                    le = ls[e]
                    ev = jnp.full((_LANES,), e, jnp.int32)
                    gt1 = le > m1
                    gt2 = le > m2
                    m2n = jnp.where(gt1, m1, jnp.where(gt2, le, m2))
                    i2n = jnp.where(gt1, i1, jnp.where(gt2, ev, i2))
                    m1 = jnp.where(gt1, le, m1)
                    i1 = jnp.where(gt1, ev, i1)
                    m2 = m2n
                    i2 = i2n
                z = jnp.exp(m2 - m1)
                s = 1.0 + z
                w1 = 1.0 / s
                w2 = z / s
                orow2 = (c * _CHUNK + row) * 2
                plsc.store_scatter(w_v, [orow2], w1)
                plsc.store_scatter(w_v, [orow2 + 1], w2)
                plsc.store_scatter(i_v, [orow2], i1)
                plsc.store_scatter(i_v, [orow2 + 1], i2)
                return _

            lax.fori_loop(0, _CHUNK // _LANES, group, 0)
            return carry

        lax.fori_loop(0, tpw // _CHUNK, chunk, 0)
        pltpu.sync_copy(w_v, w_hbm.at[pl.ds(base * 2, tpw * 2)])
        pltpu.sync_copy(i_v, i_hbm.at[pl.ds(base * 2, tpw * 2)])

    return router


def kernel(hidden_states, gate_w):
    b, s, h = hidden_states.shape
    flat = hidden_states.reshape(-1, h)
    n_tok = flat.shape[0]
    logits = _compute_logits(flat, gate_w.T)
    w_flat, i_flat = _make_router(n_tok)(logits)
    return w_flat.reshape(n_tok, 2), i_flat.reshape(n_tok, 2), logits


# speedup vs baseline: 1.2826x; 1.2293x over previous
import functools

import jax
import jax.numpy as jnp
from jax import lax
from jax.experimental import pallas as pl
from jax.experimental.pallas import tpu as pltpu
from jax.experimental.pallas import tpu_sc as plsc

_HID = 768
_NEXP = 8
_LANES = 16
_NCORES = 2
_NSUB = 16
_NWORK = _NCORES * _NSUB
_CHUNK = 256

_TC_BLK = 2048


def _logits_body(x_ref, w_ref, out_ref):
    out_ref[...] = lax.dot_general(
        x_ref[...], w_ref[...],
        dimension_numbers=(((1,), (0,)), ((), ())),
        preferred_element_type=jnp.float32)


def _compute_logits(flat, w_t):
    n_tok = flat.shape[0]
    return pl.pallas_call(
        _logits_body,
        grid=(n_tok // _TC_BLK,),
        in_specs=[
            pl.BlockSpec((_TC_BLK, _HID), lambda i: (i, 0)),
            pl.BlockSpec((_HID, _NEXP), lambda i: (0, 0)),
        ],
        out_specs=pl.BlockSpec((_TC_BLK, _NEXP), lambda i: (i, 0)),
        out_shape=jax.ShapeDtypeStruct((n_tok, _NEXP), jnp.float32),
    )(flat, w_t)


def _make_router(n_tok):
    tpw = n_tok // _NWORK

    @functools.partial(
        pl.kernel,
        mesh=plsc.VectorSubcoreMesh(core_axis_name="c", subcore_axis_name="s"),
        compiler_params=pltpu.CompilerParams(needs_layout_passes=False),
        out_type=[
            jax.ShapeDtypeStruct((n_tok, 2), jnp.float32),
            jax.ShapeDtypeStruct((n_tok, 2), jnp.int32),
        ],
        scratch_types=[
            pltpu.VMEM((_CHUNK, _NEXP), jnp.float32),
            pltpu.VMEM((_CHUNK, 2), jnp.float32),
            pltpu.VMEM((_CHUNK, 2), jnp.int32),
        ],
    )
    def router(logits_hbm, w_hbm, i_hbm, lg_v, w_v, i_v):
        wid = lax.axis_index("s") * _NCORES + lax.axis_index("c")
        base = wid * tpw
        lane = lax.iota(jnp.int32, _LANES)
        zero = jnp.zeros((_LANES,), jnp.int32)
        one = jnp.full((_LANES,), 1, jnp.int32)

        def chunk(c, carry):
            cbase = base + c * _CHUNK
            pltpu.sync_copy(logits_hbm.at[pl.ds(cbase, _CHUNK)], lg_v)

            def group(g, _):
                row = g * _LANES + lane
                ls = [
                    plsc.load_gather(
                        lg_v, [row, jnp.full((_LANES,), e, jnp.int32)])
                    for e in range(_NEXP)
                ]
                m1 = ls[0]
                i1 = jnp.zeros((_LANES,), jnp.int32)
                m2 = jnp.full((_LANES,), -jnp.inf, jnp.float32)
                i2 = jnp.zeros((_LANES,), jnp.int32)
                for e in range(1, _NEXP):
                    le = ls[e]
                    ev = jnp.full((_LANES,), e, jnp.int32)
                    gt1 = le > m1
                    gt2 = le > m2
                    m2n = jnp.where(gt1, m1, jnp.where(gt2, le, m2))
                    i2n = jnp.where(gt1, i1, jnp.where(gt2, ev, i2))
                    m1 = jnp.where(gt1, le, m1)
                    i1 = jnp.where(gt1, ev, i1)
                    m2 = m2n
                    i2 = i2n
                z = jnp.exp(m2 - m1)
                s = 1.0 + z
                w1 = 1.0 / s
                w2 = z / s
                plsc.store_scatter(w_v, [row, zero], w1)
                plsc.store_scatter(w_v, [row, one], w2)
                plsc.store_scatter(i_v, [row, zero], i1)
                plsc.store_scatter(i_v, [row, one], i2)
                return _

            lax.fori_loop(0, _CHUNK // _LANES, group, 0)
            pltpu.sync_copy(w_v, w_hbm.at[pl.ds(cbase, _CHUNK)])
            pltpu.sync_copy(i_v, i_hbm.at[pl.ds(cbase, _CHUNK)])
            return carry

        lax.fori_loop(0, tpw // _CHUNK, chunk, 0)

    return router


def kernel(hidden_states, gate_w):
    b, s, h = hidden_states.shape
    flat = hidden_states.reshape(-1, h)
    n_tok = flat.shape[0]
    logits = _compute_logits(flat, gate_w.T)
    weights, indices = _make_router(n_tok)(logits)
    return weights, indices, logits


# speedup vs baseline: 2.3207x; 1.8094x over previous
import functools

import jax
import jax.numpy as jnp
from jax import lax
from jax.experimental import pallas as pl
from jax.experimental.pallas import tpu as pltpu
from jax.experimental.pallas import tpu_sc as plsc

_HID = 768
_NEXP = 8
_LANES = 16
_NCORES = 2
_NSUB = 16
_NWORK = _NCORES * _NSUB
_TBLK = 128

_TC_BLK = 2048


def _logits_body(x_ref, w_ref, out_ref):
    logits = lax.dot_general(
        x_ref[...], w_ref[...],
        dimension_numbers=(((1,), (0,)), ((), ())),
        preferred_element_type=jnp.float32)
    t = jnp.transpose(logits)
    nblk = _TC_BLK // _TBLK
    out_ref[...] = jnp.transpose(
        t.reshape(_NEXP, nblk, _TBLK), (1, 0, 2))


def _compute_logits_t3(flat, w_t):
    n_tok = flat.shape[0]
    nblk = _TC_BLK // _TBLK
    return pl.pallas_call(
        _logits_body,
        grid=(n_tok // _TC_BLK,),
        in_specs=[
            pl.BlockSpec((_TC_BLK, _HID), lambda i: (i, 0)),
            pl.BlockSpec((_HID, _NEXP), lambda i: (0, 0)),
        ],
        out_specs=pl.BlockSpec((nblk, _NEXP, _TBLK), lambda i: (i, 0, 0)),
        out_shape=jax.ShapeDtypeStruct((n_tok // _TBLK, _NEXP, _TBLK),
                                       jnp.float32),
    )(flat, w_t)


def _make_router(n_tok):
    tpw = n_tok // _NWORK
    bpw = tpw // _TBLK

    @functools.partial(
        pl.kernel,
        mesh=plsc.VectorSubcoreMesh(core_axis_name="c", subcore_axis_name="s"),
        compiler_params=pltpu.CompilerParams(needs_layout_passes=False),
        out_type=[
            jax.ShapeDtypeStruct((n_tok * 2,), jnp.float32),
            jax.ShapeDtypeStruct((n_tok * 2,), jnp.int32),
        ],
        scratch_types=[
            pltpu.VMEM((tpw * _NEXP,), jnp.float32),
            pltpu.VMEM((tpw * 2,), jnp.float32),
            pltpu.VMEM((tpw * 2,), jnp.int32),
        ],
    )
    def router(logits_hbm, w_hbm, i_hbm, lg_v, w_v, i_v):
        wid = lax.axis_index("s") * _NCORES + lax.axis_index("c")
        pltpu.sync_copy(logits_hbm.at[pl.ds(wid * tpw * _NEXP, tpw * _NEXP)],
                        lg_v)

        def group(k, carry):
            b = k >> 3
            g = k & 7
            lbase = b * (_TBLK * _NEXP) + g * _LANES
            ls = [lg_v[pl.ds(lbase + e * _TBLK, _LANES)] for e in range(_NEXP)]
            m1 = ls[0]
            i1 = jnp.zeros((_LANES,), jnp.int32)
            m2 = jnp.full((_LANES,), -jnp.inf, jnp.float32)
            i2 = jnp.zeros((_LANES,), jnp.int32)
            for e in range(1, _NEXP):
                le = ls[e]
                ev = jnp.full((_LANES,), e, jnp.int32)
                gt1 = le > m1
                gt2 = le > m2
                m2n = jnp.where(gt1, m1, jnp.where(gt2, le, m2))
                i2n = jnp.where(gt1, i1, jnp.where(gt2, ev, i2))
                m1 = jnp.where(gt1, le, m1)
                i1 = jnp.where(gt1, ev, i1)
                m2 = m2n
                i2 = i2n
            z = jnp.exp(m2 - m1)
            s = 1.0 + z
            w1 = 1.0 / s
            w2 = z / s
            obase = b * (_TBLK * 2) + g * _LANES
            w_v[pl.ds(obase, _LANES)] = w1
            w_v[pl.ds(obase + _TBLK, _LANES)] = w2
            i_v[pl.ds(obase, _LANES)] = i1
            i_v[pl.ds(obase + _TBLK, _LANES)] = i2
            return carry

        lax.fori_loop(0, bpw * (_TBLK // _LANES), group, 0)
        pltpu.sync_copy(w_v, w_hbm.at[pl.ds(wid * tpw * 2, tpw * 2)])
        pltpu.sync_copy(i_v, i_hbm.at[pl.ds(wid * tpw * 2, tpw * 2)])

    return router


def kernel(hidden_states, gate_w):
    b, s, h = hidden_states.shape
    flat = hidden_states.reshape(-1, h)
    n_tok = flat.shape[0]
    nblk = n_tok // _TBLK
    logits_t3 = _compute_logits_t3(flat, gate_w.T)
    w_flat, i_flat = _make_router(n_tok)(logits_t3.reshape(-1))
    weights = w_flat.reshape(nblk, 2, _TBLK).transpose(0, 2, 1)
    weights = weights.reshape(n_tok, 2)
    indices = i_flat.reshape(nblk, 2, _TBLK).transpose(0, 2, 1)
    indices = indices.reshape(n_tok, 2)
    logits = logits_t3.transpose(0, 2, 1).reshape(n_tok, _NEXP)
    return weights, indices, logits


# speedup vs baseline: 2.4416x; 1.0521x over previous
import functools

import jax
import jax.numpy as jnp
from jax import lax
from jax.experimental import pallas as pl
from jax.experimental.pallas import tpu as pltpu
from jax.experimental.pallas import tpu_sc as plsc

_HID = 768
_NEXP = 8
_LANES = 16
_NCORES = 2
_NSUB = 16
_NWORK = _NCORES * _NSUB
_TBLK = 128

_TC_BLK = 2048


def _logits_body(x_ref, w_ref, out_ref):
    logits = lax.dot_general(
        x_ref[...], w_ref[...],
        dimension_numbers=(((1,), (1,)), ((), ())),
        preferred_element_type=jnp.float32)
    t = jnp.transpose(logits)
    nblk = _TC_BLK // _TBLK
    out_ref[...] = jnp.transpose(
        t.reshape(_NEXP, nblk, _TBLK), (1, 0, 2))


def _compute_logits_t3(flat, w_t):
    n_tok = flat.shape[0]
    nblk = _TC_BLK // _TBLK
    return pl.pallas_call(
        _logits_body,
        grid=(n_tok // _TC_BLK,),
        in_specs=[
            pl.BlockSpec((_TC_BLK, _HID), lambda i: (i, 0)),
            pl.BlockSpec((_NEXP, _HID), lambda i: (0, 0)),
        ],
        out_specs=pl.BlockSpec((nblk, _NEXP, _TBLK), lambda i: (i, 0, 0)),
        out_shape=jax.ShapeDtypeStruct((n_tok // _TBLK, _NEXP, _TBLK),
                                       jnp.float32),
    )(flat, w_t)


def _make_router(n_tok):
    tpw = n_tok // _NWORK
    bpw = tpw // _TBLK

    @functools.partial(
        pl.kernel,
        mesh=plsc.VectorSubcoreMesh(core_axis_name="c", subcore_axis_name="s"),
        compiler_params=pltpu.CompilerParams(needs_layout_passes=False),
        out_type=[
            jax.ShapeDtypeStruct((n_tok * 2,), jnp.float32),
            jax.ShapeDtypeStruct((n_tok * 2,), jnp.int32),
        ],
        scratch_types=[
            pltpu.VMEM((tpw * _NEXP,), jnp.float32),
            pltpu.VMEM((tpw * 2,), jnp.float32),
            pltpu.VMEM((tpw * 2,), jnp.int32),
        ],
    )
    def router(logits_hbm, w_hbm, i_hbm, lg_v, w_v, i_v):
        wid = lax.axis_index("s") * _NCORES + lax.axis_index("c")
        pltpu.sync_copy(logits_hbm.at[pl.ds(wid * tpw * _NEXP, tpw * _NEXP)],
                        lg_v)

        def group(k, carry):
            b = k >> 3
            g = k & 7
            lbase = b * (_TBLK * _NEXP) + g * _LANES
            ls = [lg_v[pl.ds(lbase + e * _TBLK, _LANES)] for e in range(_NEXP)]
            m1 = ls[0]
            i1 = jnp.zeros((_LANES,), jnp.int32)
            m2 = jnp.full((_LANES,), -jnp.inf, jnp.float32)
            i2 = jnp.zeros((_LANES,), jnp.int32)
            for e in range(1, _NEXP):
                le = ls[e]
                ev = jnp.full((_LANES,), e, jnp.int32)
                gt1 = le > m1
                gt2 = le > m2
                m2n = jnp.where(gt1, m1, jnp.where(gt2, le, m2))
                i2n = jnp.where(gt1, i1, jnp.where(gt2, ev, i2))
                m1 = jnp.where(gt1, le, m1)
                i1 = jnp.where(gt1, ev, i1)
                m2 = m2n
                i2 = i2n
            z = jnp.exp(m2 - m1)
            s = 1.0 + z
            w1 = 1.0 / s
            w2 = z / s
            obase = b * (_TBLK * 2) + g * _LANES
            w_v[pl.ds(obase, _LANES)] = w1
            w_v[pl.ds(obase + _TBLK, _LANES)] = w2
            i_v[pl.ds(obase, _LANES)] = i1
            i_v[pl.ds(obase + _TBLK, _LANES)] = i2
            return carry

        lax.fori_loop(0, bpw * (_TBLK // _LANES), group, 0)
        pltpu.sync_copy(w_v, w_hbm.at[pl.ds(wid * tpw * 2, tpw * 2)])
        pltpu.sync_copy(i_v, i_hbm.at[pl.ds(wid * tpw * 2, tpw * 2)])

    return router


def kernel(hidden_states, gate_w):
    b, s, h = hidden_states.shape
    flat = hidden_states.reshape(-1, h)
    n_tok = flat.shape[0]
    nblk = n_tok // _TBLK
    logits_t3 = _compute_logits_t3(flat, gate_w)
    w_flat, i_flat = _make_router(n_tok)(logits_t3.reshape(-1))
    weights = w_flat.reshape(nblk, 2, _TBLK).transpose(0, 2, 1)
    weights = weights.reshape(n_tok, 2)
    indices = i_flat.reshape(nblk, 2, _TBLK).transpose(0, 2, 1)
    indices = indices.reshape(n_tok, 2)
    logits = logits_t3.transpose(0, 2, 1).reshape(n_tok, _NEXP)
    return weights, indices, logits
